# baseline (speedup 1.0000x reference)
import dataclasses
import functools

import jax
import jax.numpy as jnp
from jax import lax
from jax.experimental import pallas as pl
from jax.experimental.pallas import tpu as pltpu
from jax.experimental.pallas import tpu_sc as plsc

H = 1024
EPS = 1e-5
NC, NS = 2, 16
NW = NC * NS
PCH = 16
LANES = 16
S_SPLITS = (512, 512, 1024, 1024, 1024)


def _sc_gather_pack(emb_table, ids_flat, s_full, off, s_c, b):
    rows = b * s_c
    half = s_c // 2
    pairs = rows // 2
    p_per_w = pairs // NW
    nsub = p_per_w // PCH
    w_per_b = half // p_per_w
    assert p_per_w % PCH == 0 and half % p_per_w == 0
    mesh = plsc.VectorSubcoreMesh(core_axis_name="c", subcore_axis_name="s")
    cp = pltpu.CompilerParams()
    if "needs_layout_passes" in pltpu.CompilerParams.__dataclass_fields__:
        cp = dataclasses.replace(cp, needs_layout_passes=False)

    @functools.partial(
        pl.kernel,
        out_type=jax.ShapeDtypeStruct((pairs, H), jnp.int32),
        mesh=mesh,
        compiler_params=cp,
        scratch_types=[
            pltpu.VMEM((p_per_w,), jnp.int32),
            pltpu.VMEM((p_per_w,), jnp.int32),
            pltpu.VMEM((PCH, H), jnp.float32),
            pltpu.VMEM((PCH, H), jnp.float32),
            pltpu.VMEM((PCH, H), jnp.float32),
            pltpu.VMEM((PCH, H), jnp.float32),
            pltpu.VMEM((PCH, H), jnp.int32),
            pltpu.VMEM((PCH, H), jnp.int32),
            pltpu.SemaphoreType.DMA,
            pltpu.SemaphoreType.DMA,
            pltpu.SemaphoreType.DMA,
            pltpu.SemaphoreType.DMA,
            pltpu.SemaphoreType.DMA,
            pltpu.SemaphoreType.DMA,
        ],
    )
    def gather_k(table_hbm, idx_hbm, out_hbm, ia_v, ib_v,
                 a0, a1, b0, b1, p0_, p1_,
                 sa0, sa1, sb0, sb1, sw0, sw1):
        wid = lax.axis_index("s") * NC + lax.axis_index("c")
        batch = wid // w_per_b
        p0 = (wid % w_per_b) * p_per_w
        src = batch * s_full + off + p0
        pltpu.sync_copy(idx_hbm.at[pl.ds(src, p_per_w)], ia_v)
        pltpu.sync_copy(idx_hbm.at[pl.ds(src + half, p_per_w)], ib_v)

        out_base = batch * half + p0
        abuf = (a0, a1)
        bbuf = (b0, b1)
        pbuf = (p0_, p1_)
        sa = (sa0, sa1)
        sb = (sb0, sb1)
        sw = (sw0, sw1)

        gha = [None] * nsub
        ghb = [None] * nsub
        wh = [None] * nsub
        gha[0] = pltpu.async_copy(
            table_hbm.at[ia_v.at[pl.ds(0, PCH)]], abuf[0], sa[0])
        ghb[0] = pltpu.async_copy(
            table_hbm.at[ib_v.at[pl.ds(0, PCH)]], bbuf[0], sb[0])
        for c in range(nsub):
            p = c % 2
            if c + 1 < nsub:
                gha[c + 1] = pltpu.async_copy(
                    table_hbm.at[ia_v.at[pl.ds((c + 1) * PCH, PCH)]],
                    abuf[1 - p], sa[1 - p])
                ghb[c + 1] = pltpu.async_copy(
                    table_hbm.at[ib_v.at[pl.ds((c + 1) * PCH, PCH)]],
                    bbuf[1 - p], sb[1 - p])
            gha[c].wait()
            ghb[c].wait()
            if c >= 2:
                wh[c - 2].wait()
            ra = abuf[p]
            rb_ = bbuf[p]
            rp = pbuf[p]

            @pl.loop(0, H // LANES)
            def _(g):
                sl = pl.ds(g * LANES, LANES)
                for r in range(PCH):
                    packed = plsc.pack(
                        ra[r, sl], rb_[r, sl],
                        format=plsc.PackFormat.INTERLEAVED)
                    rp[r, sl] = plsc.bitcast(packed, jnp.int32)

            wh[c] = pltpu.async_copy(
                rp, out_hbm.at[pl.ds(out_base + c * PCH, PCH)], sw[p])
        if nsub >= 2:
            wh[nsub - 2].wait()
        wh[nsub - 1].wait()

    return gather_k(emb_table, ids_flat)


def _ln_math(x, g, b_, o_ref, sl):
    mean = jnp.mean(x, axis=1, keepdims=True)
    xc = x - mean
    var = jnp.mean(xc * xc, axis=1, keepdims=True)
    inv = lax.rsqrt(var + EPS)
    o_ref[sl, :] = (xc * inv) * g + b_


def _ln_body_first(tok_ref, pos_ref, g_ref, b_ref, o_ref):
    w = tok_ref[...]
    half = w.shape[0]
    a = lax.bitcast_convert_type(lax.shift_left(w, 16), jnp.float32)
    bb = lax.bitcast_convert_type(
        jnp.bitwise_and(w, jnp.int32(-65536)), jnp.float32)
    g = g_ref[...]
    be = b_ref[...]
    _ln_math(a + pos_ref[0:half, :], g, be, o_ref, slice(0, half))
    _ln_math(bb + pos_ref[half:2 * half, :], g, be, o_ref,
             slice(half, 2 * half))


def _ln_body_chained(acc_ref, tok_ref, pos_ref, g_ref, b_ref, o_ref):
    del acc_ref
    _ln_body_first(tok_ref, pos_ref, g_ref, b_ref, o_ref)


def _tc_layernorm_chunk(acc, tok, pos, gamma, beta, b, s, s_c, off):
    half = s_c // 2
    sposb = s // s_c
    offb = off // s_c
    grid = (b,)
    tok_spec = pl.BlockSpec((half, H), lambda j: (j, 0))
    pos_spec = pl.BlockSpec((s_c, H), lambda j: (offb, 0))
    vec_spec = pl.BlockSpec((1, H), lambda j: (0, 0))
    out_spec = pl.BlockSpec((s_c, H), lambda j: (j * sposb + offb, 0))
    out_shape = jax.ShapeDtypeStruct((b * s, H), jnp.float32)
    if acc is None:
        return pl.pallas_call(
            _ln_body_first,
            grid=grid,
            in_specs=[tok_spec, pos_spec, vec_spec, vec_spec],
            out_specs=out_spec,
            out_shape=out_shape,
        )(tok, pos, gamma, beta)
    acc_spec = pl.BlockSpec((8, 128), lambda j: (0, 0))
    return pl.pallas_call(
        _ln_body_chained,
        grid=grid,
        in_specs=[acc_spec, tok_spec, pos_spec, vec_spec, vec_spec],
        out_specs=out_spec,
        out_shape=out_shape,
        input_output_aliases={0: 0},
    )(acc, tok, pos, gamma, beta)


def kernel(input_ids, emb_table, pos_table, gamma, beta):
    b, s = input_ids.shape
    ids = input_ids.astype(jnp.int32)
    g2 = gamma.reshape(1, H)
    b2 = beta.reshape(1, H)

    offs = [0]
    for s_c in S_SPLITS:
        offs.append(offs[-1] + s_c)
    assert offs[-1] == s

    ids_flat = ids.reshape(b * s)
    toks = [
        _sc_gather_pack(emb_table, ids_flat, s, offs[k], s_c, b)
        for k, s_c in enumerate(S_SPLITS)
    ]
    acc = None
    for k, s_c in enumerate(S_SPLITS):
        acc = _tc_layernorm_chunk(
            acc, toks[k], pos_table, g2, b2, b, s, s_c, offs[k])
    return acc.reshape(b, s, H)


# speedup vs baseline: 1.1042x; 1.1042x over previous
import dataclasses
import functools

import jax
import jax.numpy as jnp
from jax import lax
from jax.experimental import pallas as pl
from jax.experimental.pallas import tpu as pltpu
from jax.experimental.pallas import tpu_sc as plsc

H = 1024
EPS = 1e-5
NC, NS = 2, 16
NW = NC * NS
PCH = 16
LANES = 16
S_SPLITS = (1024, 1024, 2048)


def _sc_gather_pack(emb_table, ids_flat, s_full, off, s_c, b):
    rows = b * s_c
    half = s_c // 2
    pairs = rows // 2
    p_per_w = pairs // NW
    nsub = p_per_w // PCH
    w_per_b = half // p_per_w
    assert p_per_w % PCH == 0 and half % p_per_w == 0
    mesh = plsc.VectorSubcoreMesh(core_axis_name="c", subcore_axis_name="s")
    cp = pltpu.CompilerParams()
    if "needs_layout_passes" in pltpu.CompilerParams.__dataclass_fields__:
        cp = dataclasses.replace(cp, needs_layout_passes=False)

    @functools.partial(
        pl.kernel,
        out_type=jax.ShapeDtypeStruct((pairs, H), jnp.int32),
        mesh=mesh,
        compiler_params=cp,
        scratch_types=[
            pltpu.VMEM((p_per_w,), jnp.int32),
            pltpu.VMEM((p_per_w,), jnp.int32),
            pltpu.VMEM((PCH, H), jnp.float32),
            pltpu.VMEM((PCH, H), jnp.float32),
            pltpu.VMEM((PCH, H), jnp.float32),
            pltpu.VMEM((PCH, H), jnp.float32),
            pltpu.VMEM((PCH, H), jnp.int32),
            pltpu.VMEM((PCH, H), jnp.int32),
            pltpu.SemaphoreType.DMA,
            pltpu.SemaphoreType.DMA,
            pltpu.SemaphoreType.DMA,
            pltpu.SemaphoreType.DMA,
            pltpu.SemaphoreType.DMA,
            pltpu.SemaphoreType.DMA,
        ],
    )
    def gather_k(table_hbm, idx_hbm, out_hbm, ia_v, ib_v,
                 a0, a1, b0, b1, p0_, p1_,
                 sa0, sa1, sb0, sb1, sw0, sw1):
        wid = lax.axis_index("s") * NC + lax.axis_index("c")
        batch = wid // w_per_b
        p0 = (wid % w_per_b) * p_per_w
        src = batch * s_full + off + p0
        pltpu.sync_copy(idx_hbm.at[pl.ds(src, p_per_w)], ia_v)
        pltpu.sync_copy(idx_hbm.at[pl.ds(src + half, p_per_w)], ib_v)

        out_base = batch * half + p0
        abuf = (a0, a1)
        bbuf = (b0, b1)
        pbuf = (p0_, p1_)
        sa = (sa0, sa1)
        sb = (sb0, sb1)
        sw = (sw0, sw1)

        gha = [None] * nsub
        ghb = [None] * nsub
        wh = [None] * nsub
        gha[0] = pltpu.async_copy(
            table_hbm.at[ia_v.at[pl.ds(0, PCH)]], abuf[0], sa[0])
        ghb[0] = pltpu.async_copy(
            table_hbm.at[ib_v.at[pl.ds(0, PCH)]], bbuf[0], sb[0])
        for c in range(nsub):
            p = c % 2
            if c + 1 < nsub:
                gha[c + 1] = pltpu.async_copy(
                    table_hbm.at[ia_v.at[pl.ds((c + 1) * PCH, PCH)]],
                    abuf[1 - p], sa[1 - p])
                ghb[c + 1] = pltpu.async_copy(
                    table_hbm.at[ib_v.at[pl.ds((c + 1) * PCH, PCH)]],
                    bbuf[1 - p], sb[1 - p])
            gha[c].wait()
            ghb[c].wait()
            if c >= 2:
                wh[c - 2].wait()
            ra = abuf[p]
            rb_ = bbuf[p]
            rp = pbuf[p]

            @pl.loop(0, H // LANES)
            def _(g):
                sl = pl.ds(g * LANES, LANES)
                for r in range(PCH):
                    packed = plsc.pack(
                        ra[r, sl], rb_[r, sl],
                        format=plsc.PackFormat.INTERLEAVED)
                    rp[r, sl] = plsc.bitcast(packed, jnp.int32)

            wh[c] = pltpu.async_copy(
                rp, out_hbm.at[pl.ds(out_base + c * PCH, PCH)], sw[p])
        if nsub >= 2:
            wh[nsub - 2].wait()
        wh[nsub - 1].wait()

    return gather_k(emb_table, ids_flat)


def _ln_math(x, g, b_, o_ref, sl):
    mean = jnp.mean(x, axis=1, keepdims=True)
    xc = x - mean
    var = jnp.mean(xc * xc, axis=1, keepdims=True)
    inv = lax.rsqrt(var + EPS)
    o_ref[sl, :] = (xc * inv) * g + b_


def _ln_body_first(tok_ref, pos_ref, g_ref, b_ref, o_ref):
    w = tok_ref[...]
    half = w.shape[0]
    a = lax.bitcast_convert_type(lax.shift_left(w, 16), jnp.float32)
    bb = lax.bitcast_convert_type(
        jnp.bitwise_and(w, jnp.int32(-65536)), jnp.float32)
    g = g_ref[...]
    be = b_ref[...]
    _ln_math(a + pos_ref[0:half, :], g, be, o_ref, slice(0, half))
    _ln_math(bb + pos_ref[half:2 * half, :], g, be, o_ref,
             slice(half, 2 * half))


def _ln_body_chained(acc_ref, tok_ref, pos_ref, g_ref, b_ref, o_ref):
    del acc_ref
    _ln_body_first(tok_ref, pos_ref, g_ref, b_ref, o_ref)


def _tc_layernorm_chunk(acc, tok, pos, gamma, beta, b, s, s_c, off):
    half = s_c // 2
    sposb = s // s_c
    offb = off // s_c
    grid = (b,)
    tok_spec = pl.BlockSpec((half, H), lambda j: (j, 0))
    pos_spec = pl.BlockSpec((s_c, H), lambda j: (offb, 0))
    vec_spec = pl.BlockSpec((1, H), lambda j: (0, 0))
    out_spec = pl.BlockSpec((s_c, H), lambda j: (j * sposb + offb, 0))
    out_shape = jax.ShapeDtypeStruct((b * s, H), jnp.float32)
    if acc is None:
        return pl.pallas_call(
            _ln_body_first,
            grid=grid,
            in_specs=[tok_spec, pos_spec, vec_spec, vec_spec],
            out_specs=out_spec,
            out_shape=out_shape,
        )(tok, pos, gamma, beta)
    acc_spec = pl.BlockSpec((8, 128), lambda j: (0, 0))
    return pl.pallas_call(
        _ln_body_chained,
        grid=grid,
        in_specs=[acc_spec, tok_spec, pos_spec, vec_spec, vec_spec],
        out_specs=out_spec,
        out_shape=out_shape,
        input_output_aliases={0: 0},
    )(acc, tok, pos, gamma, beta)


def kernel(input_ids, emb_table, pos_table, gamma, beta):
    b, s = input_ids.shape
    ids = input_ids.astype(jnp.int32)
    g2 = gamma.reshape(1, H)
    b2 = beta.reshape(1, H)

    offs = [0]
    for s_c in S_SPLITS:
        offs.append(offs[-1] + s_c)
    assert offs[-1] == s

    ids_flat = ids.reshape(b * s)
    toks = [
        _sc_gather_pack(emb_table, ids_flat, s, offs[k], s_c, b)
        for k, s_c in enumerate(S_SPLITS)
    ]
    acc = None
    for k, s_c in enumerate(S_SPLITS):
        acc = _tc_layernorm_chunk(
            acc, toks[k], pos_table, g2, b2, b, s, s_c, offs[k])
    return acc.reshape(b, s, H)
